# baseline (speedup 1.0000x reference)
import jax
import jax.numpy as jnp
from jax.experimental import pallas as pl

NUM_NODES = 10000
NODE_DIM = 128
NUM_NETS = 4096

M_BLOCK = 256


def _mean_net_kernel(mask_ref, emb_ref, out_ref):
    m = mask_ref[...]
    mb = m.astype(jnp.bfloat16)
    sums = jax.lax.dot_general(
        mb, emb_ref[...],
        dimension_numbers=(((1,), (0,)), ((), ())),
        preferred_element_type=jnp.float32,
    )
    counts = jnp.sum(m.astype(jnp.float32), axis=1, keepdims=True)
    out_ref[...] = jnp.where(counts > 0, sums / jnp.maximum(counts, 1.0), 0.0)


def kernel(node_embeddings, attention_mask):
    emb_bf16 = node_embeddings.astype(jnp.bfloat16)
    grid = (NUM_NETS // M_BLOCK,)
    return pl.pallas_call(
        _mean_net_kernel,
        grid=grid,
        in_specs=[
            pl.BlockSpec((M_BLOCK, NUM_NODES), lambda i: (i, 0)),
            pl.BlockSpec((NUM_NODES, NODE_DIM), lambda i: (0, 0)),
        ],
        out_specs=pl.BlockSpec((M_BLOCK, NODE_DIM), lambda i: (i, 0)),
        out_shape=jax.ShapeDtypeStruct((NUM_NETS, NODE_DIM), jnp.float32),
    )(attention_mask, emb_bf16)


# speedup vs baseline: 1.6271x; 1.6271x over previous
import jax
import jax.numpy as jnp
from jax.experimental import pallas as pl

NUM_NODES = 10000
NODE_DIM = 128
NUM_NETS = 4096

M_BLOCK = 256


def _mean_net_kernel(mask_ref, emb_ref, out_ref):
    m = mask_ref[...] != 0
    mb = m.astype(jnp.bfloat16)
    sums = jax.lax.dot_general(
        mb, emb_ref[...],
        dimension_numbers=(((1,), (0,)), ((), ())),
        preferred_element_type=jnp.float32,
    )
    counts = jnp.sum(m.astype(jnp.float32), axis=1, keepdims=True)
    out_ref[...] = jnp.where(counts > 0, sums / jnp.maximum(counts, 1.0), 0.0)


def kernel(node_embeddings, attention_mask):
    emb_bf16 = node_embeddings.astype(jnp.bfloat16)
    mask_i8 = attention_mask.view(jnp.int8)
    grid = (NUM_NETS // M_BLOCK,)
    return pl.pallas_call(
        _mean_net_kernel,
        grid=grid,
        in_specs=[
            pl.BlockSpec((M_BLOCK, NUM_NODES), lambda i: (i, 0)),
            pl.BlockSpec((NUM_NODES, NODE_DIM), lambda i: (0, 0)),
        ],
        out_specs=pl.BlockSpec((M_BLOCK, NODE_DIM), lambda i: (i, 0)),
        out_shape=jax.ShapeDtypeStruct((NUM_NETS, NODE_DIM), jnp.float32),
    )(mask_i8, emb_bf16)


# speedup vs baseline: 2.4705x; 1.5184x over previous
import jax
import jax.numpy as jnp
from jax.experimental import pallas as pl

NUM_NODES = 10000
NODE_DIM = 128
NUM_NETS = 4096

M_BLOCK = 256


def _mean_net_kernel(mask_ref, emb_ref, out_ref):
    mb = mask_ref[...].astype(jnp.bfloat16)
    sums = jax.lax.dot_general(
        mb, emb_ref[...],
        dimension_numbers=(((1,), (0,)), ((), ())),
        preferred_element_type=jnp.float32,
    )
    counts = jnp.sum(mb, axis=1, keepdims=True, dtype=jnp.float32)
    out_ref[...] = jnp.where(counts > 0, sums / jnp.maximum(counts, 1.0), 0.0)


def kernel(node_embeddings, attention_mask):
    emb_bf16 = node_embeddings.astype(jnp.bfloat16)
    mask_i8 = attention_mask.astype(jnp.int8)
    grid = (NUM_NETS // M_BLOCK,)
    return pl.pallas_call(
        _mean_net_kernel,
        grid=grid,
        in_specs=[
            pl.BlockSpec((M_BLOCK, NUM_NODES), lambda i: (i, 0)),
            pl.BlockSpec((NUM_NODES, NODE_DIM), lambda i: (0, 0)),
        ],
        out_specs=pl.BlockSpec((M_BLOCK, NODE_DIM), lambda i: (i, 0)),
        out_shape=jax.ShapeDtypeStruct((NUM_NETS, NODE_DIM), jnp.float32),
    )(mask_i8, emb_bf16)


# speedup vs baseline: 2.4722x; 1.0007x over previous
import jax
import jax.numpy as jnp
from jax.experimental import pallas as pl
from jax.experimental.pallas import tpu as pltpu

NUM_NODES = 10000
NODE_DIM = 128
NUM_NETS = 4096

M_BLOCK = 256


def _mean_net_kernel(mask_ref, emb_ref, out_ref):
    mb = mask_ref[...].astype(jnp.bfloat16)
    sums = jax.lax.dot_general(
        mb, emb_ref[...],
        dimension_numbers=(((1,), (0,)), ((), ())),
        preferred_element_type=jnp.float32,
    )
    counts = jnp.sum(mb, axis=1, keepdims=True, dtype=jnp.float32)
    out_ref[...] = jnp.where(counts > 0, sums / jnp.maximum(counts, 1.0), 0.0)


def kernel(node_embeddings, attention_mask):
    emb_bf16 = node_embeddings.astype(jnp.bfloat16)
    mask_i8 = attention_mask.astype(jnp.int8)
    grid = (NUM_NETS // M_BLOCK,)
    return pl.pallas_call(
        _mean_net_kernel,
        grid=grid,
        in_specs=[
            pl.BlockSpec((M_BLOCK, NUM_NODES), lambda i: (i, 0)),
            pl.BlockSpec((NUM_NODES, NODE_DIM), lambda i: (0, 0)),
        ],
        out_specs=pl.BlockSpec((M_BLOCK, NODE_DIM), lambda i: (i, 0)),
        out_shape=jax.ShapeDtypeStruct((NUM_NETS, NODE_DIM), jnp.float32),
        compiler_params=pltpu.CompilerParams(
            dimension_semantics=("parallel",),
            allow_input_fusion=[True, True],
        ),
    )(mask_i8, emb_bf16)
